# Initial kernel scaffold; baseline (speedup 1.0000x reference)
import jax
import jax.numpy as jnp
from jax.experimental import pallas as pl


def kernel(encoder_hidden_states, durations_gt):
    raise NotImplementedError("write your pallas kernel here")


# speedup vs baseline: 125.6438x; 125.6438x over previous
import functools

import jax
import jax.numpy as jnp
from jax.experimental import pallas as pl

B, L, H, T = 16, 512, 256, 4096
TB = 1024


def _lr_kernel(dur_ref, enc_ref, out_ref, mask_ref):
    t0 = pl.program_id(1) * TB
    dur_row = dur_ref[0].astype(jnp.float32)

    r = jax.lax.broadcasted_iota(jnp.int32, (L, L), 0)
    c = jax.lax.broadcasted_iota(jnp.int32, (L, L), 1)
    tri_le = (c <= r).astype(jnp.float32)
    tri_lt = (c < r).astype(jnp.float32)
    dn_rhs_t = (((1,), (1,)), ((), ()))
    cum_col = jax.lax.dot_general(tri_le, dur_row, dn_rhs_t,
                                  preferred_element_type=jnp.float32)
    cum_prev_col = jax.lax.dot_general(tri_lt, dur_row, dn_rhs_t,
                                       preferred_element_type=jnp.float32)
    cum_col = cum_col.astype(jnp.int32)
    cum_prev_col = cum_prev_col.astype(jnp.int32)

    tv = t0 + jax.lax.broadcasted_iota(jnp.int32, (1, TB), 1)
    m = ((cum_prev_col <= tv) & (tv < cum_col)).astype(jnp.bfloat16)

    enc = enc_ref[0].astype(jnp.bfloat16)
    dn_both_0 = (((0,), (0,)), ((), ()))
    out_ref[0] = jax.lax.dot_general(enc, m, dn_both_0,
                                     preferred_element_type=jnp.float32)

    real_len = cum_col[L - 1:L, :]
    mask_ref[0] = (tv < real_len).astype(jnp.float32)


@jax.jit
def kernel(encoder_hidden_states, durations_gt):
    dur3 = durations_gt.reshape(B, 1, L)
    grid = (B, T // TB)
    out, mask3 = pl.pallas_call(
        _lr_kernel,
        grid=grid,
        in_specs=[
            pl.BlockSpec((1, 1, L), lambda b, t: (b, 0, 0)),
            pl.BlockSpec((1, L, H), lambda b, t: (b, 0, 0)),
        ],
        out_specs=[
            pl.BlockSpec((1, H, TB), lambda b, t: (b, 0, t)),
            pl.BlockSpec((1, 1, TB), lambda b, t: (b, 0, t)),
        ],
        out_shape=[
            jax.ShapeDtypeStruct((B, H, T), jnp.float32),
            jax.ShapeDtypeStruct((B, 1, T), jnp.float32),
        ],
    )(dur3, encoder_hidden_states)
    return out, mask3.reshape(B, T)


# speedup vs baseline: 165.6450x; 1.3184x over previous
import functools

import jax
import jax.numpy as jnp
from jax.experimental import pallas as pl

B, L, H, T = 16, 512, 256, 4096
TB = 2048


def _lr_kernel(dur_ref, enc_ref, out_ref, mask_ref):
    t0 = pl.program_id(1) * TB
    dur_row = dur_ref[0].astype(jnp.float32)

    r = jax.lax.broadcasted_iota(jnp.int32, (L, L), 0)
    c = jax.lax.broadcasted_iota(jnp.int32, (L, L), 1)
    tri_le = (c <= r).astype(jnp.float32)
    tri_lt = (c < r).astype(jnp.float32)
    dn_rhs_t = (((1,), (1,)), ((), ()))
    cum_col = jax.lax.dot_general(tri_le, dur_row, dn_rhs_t,
                                  preferred_element_type=jnp.float32)
    cum_prev_col = jax.lax.dot_general(tri_lt, dur_row, dn_rhs_t,
                                       preferred_element_type=jnp.float32)
    cum_col = cum_col.astype(jnp.int32)
    cum_prev_col = cum_prev_col.astype(jnp.int32)

    tv = t0 + jax.lax.broadcasted_iota(jnp.int32, (1, TB), 1)
    m = ((cum_prev_col <= tv) & (tv < cum_col)).astype(jnp.bfloat16)

    enc = enc_ref[0].astype(jnp.bfloat16)
    dn_both_0 = (((0,), (0,)), ((), ()))
    out_ref[0] = jax.lax.dot_general(enc, m, dn_both_0,
                                     preferred_element_type=jnp.float32)

    real_len = cum_col[L - 1:L, :]
    mask_ref[0] = (tv < real_len).astype(jnp.float32)


@jax.jit
def kernel(encoder_hidden_states, durations_gt):
    dur3 = durations_gt.reshape(B, 1, L)
    grid = (B, T // TB)
    out, mask3 = pl.pallas_call(
        _lr_kernel,
        grid=grid,
        in_specs=[
            pl.BlockSpec((1, 1, L), lambda b, t: (b, 0, 0)),
            pl.BlockSpec((1, L, H), lambda b, t: (b, 0, 0)),
        ],
        out_specs=[
            pl.BlockSpec((1, H, TB), lambda b, t: (b, 0, t)),
            pl.BlockSpec((1, 1, TB), lambda b, t: (b, 0, t)),
        ],
        out_shape=[
            jax.ShapeDtypeStruct((B, H, T), jnp.float32),
            jax.ShapeDtypeStruct((B, 1, T), jnp.float32),
        ],
    )(dur3, encoder_hidden_states)
    return out, mask3.reshape(B, T)


# speedup vs baseline: 192.4400x; 1.1618x over previous
import functools

import jax
import jax.numpy as jnp
from jax.experimental import pallas as pl

B, L, H, T = 16, 512, 256, 4096
TB = 4096


def _lr_kernel(dur_ref, enc_ref, out_ref, mask_ref):
    t0 = pl.program_id(1) * TB
    dur_row = dur_ref[0].astype(jnp.float32)

    r = jax.lax.broadcasted_iota(jnp.int32, (L, L), 0)
    c = jax.lax.broadcasted_iota(jnp.int32, (L, L), 1)
    tri_le = (c <= r).astype(jnp.float32)
    tri_lt = (c < r).astype(jnp.float32)
    dn_rhs_t = (((1,), (1,)), ((), ()))
    cum_col = jax.lax.dot_general(tri_le, dur_row, dn_rhs_t,
                                  preferred_element_type=jnp.float32)
    cum_prev_col = jax.lax.dot_general(tri_lt, dur_row, dn_rhs_t,
                                       preferred_element_type=jnp.float32)
    cum_col = cum_col.astype(jnp.int32)
    cum_prev_col = cum_prev_col.astype(jnp.int32)

    tv = t0 + jax.lax.broadcasted_iota(jnp.int32, (1, TB), 1)
    m = ((cum_prev_col <= tv) & (tv < cum_col)).astype(jnp.bfloat16)

    enc = enc_ref[0].astype(jnp.bfloat16)
    dn_both_0 = (((0,), (0,)), ((), ()))
    out_ref[0] = jax.lax.dot_general(enc, m, dn_both_0,
                                     preferred_element_type=jnp.float32)

    real_len = cum_col[L - 1:L, :]
    mask_ref[0] = (tv < real_len).astype(jnp.float32)


@jax.jit
def kernel(encoder_hidden_states, durations_gt):
    dur3 = durations_gt.reshape(B, 1, L)
    grid = (B, T // TB)
    out, mask3 = pl.pallas_call(
        _lr_kernel,
        grid=grid,
        in_specs=[
            pl.BlockSpec((1, 1, L), lambda b, t: (b, 0, 0)),
            pl.BlockSpec((1, L, H), lambda b, t: (b, 0, 0)),
        ],
        out_specs=[
            pl.BlockSpec((1, H, TB), lambda b, t: (b, 0, t)),
            pl.BlockSpec((1, 1, TB), lambda b, t: (b, 0, t)),
        ],
        out_shape=[
            jax.ShapeDtypeStruct((B, H, T), jnp.float32),
            jax.ShapeDtypeStruct((B, 1, T), jnp.float32),
        ],
    )(dur3, encoder_hidden_states)
    return out, mask3.reshape(B, T)
